# baseline (speedup 1.0000x reference)
import functools

import jax
import jax.numpy as jnp
from jax import lax
from jax.experimental import pallas as pl
from jax.experimental.pallas import tpu as pltpu
from jax.experimental.pallas import tpu_sc as plsc

B = 1024
F = 1024
OUT = 256
R = 16
T = 8
L = 16
NTILES = 32
ROWS = B // NTILES


def _sc_body(x_hbm, w_hbm, out_hbm, xbuf, wbuf, col, obuf, sem_x, sem_w):
    wid = lax.axis_index("s") * 2 + lax.axis_index("c")
    base_row = wid * ROWS

    NCH = 4
    CW = F // NCH
    cps = [
        pltpu.async_copy(
            x_hbm.at[pl.ds(base_row, ROWS), pl.ds(c * CW, CW)],
            xbuf.at[:, pl.ds(c * CW, CW)],
            sem_x.at[c],
        )
        for c in range(NCH)
    ]
    cp_w = pltpu.async_copy(w_hbm, wbuf, sem_w)

    col[pl.ds(0, L)] = jnp.full((L,), -1, jnp.int32)

    def pack_body(fg, carry):
        acc = jnp.zeros((L,), jnp.int32)
        for b in range(ROWS):
            acc = acc | (xbuf[b, pl.ds(fg * L, L)] << b)
        col[pl.ds(1 + fg * L, L)] = acc
        col[pl.ds(1 + F + fg * L, L)] = ~acc
        return carry

    for c in range(NCH):
        cps[c].wait()
        lax.fori_loop(c * (CW // L), (c + 1) * (CW // L), pack_body, 0)

    cp_w.wait()

    inv = (0, 2, 1, 3)
    iota512 = lax.iota(jnp.int32, L) * (4 * R * T)

    def og_body(og, carry):
        s = []
        for c in range(4):
            wbase = iota512 + (og * 64 + inv[c]) * (R * T)

            def r_body(r, or_acc):
                acc = jnp.full((L,), -1, jnp.int32)
                nz = jnp.zeros((L,), jnp.int32)
                for t in range(T):
                    iv = plsc.load_gather(wbuf, [wbase + (r * T + t)])
                    acc = acc & plsc.load_gather(col, [iv])
                    nz = nz | iv
                return or_acc | jnp.where(nz != 0, acc, 0)

            s.append(lax.fori_loop(0, R, r_body, jnp.zeros((L,), jnp.int32),
                                   unroll=2))
        for b in range(ROWS):
            bits = [(sc >> b) & 1 for sc in s]
            t0 = plsc.pack(bits[0], bits[1], format=plsc.PackFormat.INTERLEAVED)
            t1 = plsc.pack(bits[2], bits[3], format=plsc.PackFormat.INTERLEAVED)
            u = plsc.pack(t0, t1, format=plsc.PackFormat.INTERLEAVED,
                          preferred_element_type=jnp.int8)
            obuf[b, pl.ds(og * L, L)] = plsc.bitcast(u, jnp.int32)
        return carry

    lax.fori_loop(0, OUT // 64, og_body, 0)

    pltpu.sync_copy(obuf, out_hbm.at[pl.ds(base_row, ROWS)])


def kernel(x, weights):
    w_flat = weights.reshape(-1)
    mesh = plsc.VectorSubcoreMesh(core_axis_name="c", subcore_axis_name="s")
    f = functools.partial(
        pl.kernel,
        out_type=jax.ShapeDtypeStruct((B, OUT // 4), jnp.int32),
        mesh=mesh,
        compiler_params=pltpu.CompilerParams(needs_layout_passes=False),
        scratch_types=[
            pltpu.VMEM((ROWS, F), jnp.int32),
            pltpu.VMEM((OUT * R * T, ), jnp.int32),
            pltpu.VMEM((1 + 2 * F + 15, ), jnp.int32),
            pltpu.VMEM((ROWS, OUT // 4), jnp.int32),
            pltpu.SemaphoreType.DMA((4,)),
            pltpu.SemaphoreType.DMA,
        ],
    )(_sc_body)
    o32 = f(x, w_flat)
    shifts = jnp.arange(0, 32, 8, dtype=jnp.int32)
    bits = (o32[:, :, None] >> shifts[None, None, :]) & 1
    return bits.astype(bool).reshape(B, OUT)


# speedup vs baseline: 1.4728x; 1.4728x over previous
import functools

import jax
import jax.numpy as jnp
from jax import lax
from jax.experimental import pallas as pl
from jax.experimental.pallas import tpu as pltpu
from jax.experimental.pallas import tpu_sc as plsc

B = 1024
F = 1024
OUT = 256
R = 16
T = 8
L = 16
NTILES = 32
ROWS = B // NTILES


def _sc_body(x_hbm, w_hbm, out_hbm, xbuf, wbuf, col, obuf, sem_x, sem_w):
    wid = lax.axis_index("s") * 2 + lax.axis_index("c")
    base_row = wid * ROWS

    NCH = 4
    CW = F // NCH
    cps = [
        pltpu.async_copy(
            x_hbm.at[pl.ds(base_row, ROWS), pl.ds(c * CW, CW)],
            xbuf.at[:, pl.ds(c * CW, CW)],
            sem_x.at[c],
        )
        for c in range(NCH)
    ]
    cp_w = pltpu.async_copy(w_hbm, wbuf, sem_w)

    col[pl.ds(0, L)] = jnp.full((L,), -1, jnp.int32)

    def pack_body(fg, carry):
        acc = jnp.zeros((L,), jnp.int32)
        for b in range(ROWS):
            acc = acc | (xbuf[b, pl.ds(fg * L, L)] << b)
        col[pl.ds(1 + fg * L, L)] = acc
        col[pl.ds(1 + F + fg * L, L)] = ~acc
        return carry

    for c in range(NCH):
        cps[c].wait()
        lax.fori_loop(c * (CW // L), (c + 1) * (CW // L), pack_body, 0,
                      unroll=2)

    cp_w.wait()

    def og_body(og, carry):
        obase = og * L

        def r_body(r, or_acc):
            acc = jnp.full((L,), -1, jnp.int32)
            nz = jnp.zeros((L,), jnp.int32)
            for tp in range(T // 2):
                ab = plsc.bitcast(
                    wbuf[pl.ds(r * (T // 2) * OUT + tp * OUT + obase, L)],
                    jnp.int16)
                ia, ib = plsc.unpack(ab, format=plsc.PackFormat.INTERLEAVED)
                acc = acc & plsc.load_gather(col, [ia])
                acc = acc & plsc.load_gather(col, [ib])
                nz = nz | ia | ib
            return or_acc | jnp.where(nz != 0, acc, 0)

        or_acc = lax.fori_loop(0, R, r_body, jnp.zeros((L,), jnp.int32),
                               unroll=4)
        for b in range(ROWS):
            obuf[b, pl.ds(obase, L)] = (or_acc >> b) & 1
        return carry

    lax.fori_loop(0, OUT // L, og_body, 0)

    pltpu.sync_copy(obuf, out_hbm.at[pl.ds(base_row, ROWS)])


def kernel(x, weights):
    w_t = jnp.transpose(weights, (1, 2, 0))
    w_t = w_t.reshape(R, T // 2, 2, OUT)
    w16 = (w_t[:, :, 0, :] | (w_t[:, :, 1, :] << 16)).reshape(-1)
    mesh = plsc.VectorSubcoreMesh(core_axis_name="c", subcore_axis_name="s")
    f = functools.partial(
        pl.kernel,
        out_type=jax.ShapeDtypeStruct((B, OUT), jnp.int32),
        mesh=mesh,
        compiler_params=pltpu.CompilerParams(needs_layout_passes=False),
        scratch_types=[
            pltpu.VMEM((ROWS, F), jnp.int32),
            pltpu.VMEM((R * (T // 2) * OUT, ), jnp.int32),
            pltpu.VMEM((1 + 2 * F + 15, ), jnp.int32),
            pltpu.VMEM((ROWS, OUT), jnp.int32),
            pltpu.SemaphoreType.DMA((4,)),
            pltpu.SemaphoreType.DMA,
        ],
    )(_sc_body)
    return f(x, w16).astype(bool)


# speedup vs baseline: 1.5282x; 1.0376x over previous
import functools

import jax
import jax.numpy as jnp
from jax import lax
from jax.experimental import pallas as pl
from jax.experimental.pallas import tpu as pltpu
from jax.experimental.pallas import tpu_sc as plsc

B = 1024
F = 1024
OUT = 256
R = 16
T = 8
L = 16
NTILES = 32
ROWS = B // NTILES


def _sc_body(x_hbm, w_hbm, out_hbm, xbuf, wbuf, col, obuf, sem_x, sem_w):
    wid = lax.axis_index("s") * 2 + lax.axis_index("c")
    base_row = wid * ROWS

    NCH = 4
    CW = F // NCH
    cps = [
        pltpu.async_copy(
            x_hbm.at[pl.ds(base_row, ROWS), pl.ds(c * CW, CW)],
            xbuf.at[:, pl.ds(c * CW, CW)],
            sem_x.at[c],
        )
        for c in range(NCH)
    ]
    cp_w = pltpu.async_copy(w_hbm, wbuf, sem_w)

    col[pl.ds(0, L)] = jnp.full((L,), -1, jnp.int32)

    def pack_body(fg, carry):
        acc = jnp.zeros((L,), jnp.int32)
        for b in range(ROWS):
            acc = acc | (xbuf[b, pl.ds(fg * L, L)] << b)
        col[pl.ds(1 + fg * L, L)] = acc
        col[pl.ds(1 + F + fg * L, L)] = ~acc
        return carry

    for c in range(NCH):
        cps[c].wait()
        lax.fori_loop(c * (CW // L), (c + 1) * (CW // L), pack_body, 0)

    cp_w.wait()

    def og_body(og, carry):
        obase = og * L

        def r_body(r, or_acc):
            acc = jnp.full((L,), -1, jnp.int32)
            nz = jnp.zeros((L,), jnp.int32)
            for tp in range(T // 2):
                ab = plsc.bitcast(
                    wbuf[pl.ds(r * (T // 2) * OUT + tp * OUT + obase, L)],
                    jnp.int16)
                ia, ib = plsc.unpack(ab, format=plsc.PackFormat.INTERLEAVED)
                acc = acc & plsc.load_gather(col, [ia])
                acc = acc & plsc.load_gather(col, [ib])
                nz = nz | ia | ib
            return or_acc | jnp.where(nz != 0, acc, 0)

        or_acc = lax.fori_loop(0, R, r_body, jnp.zeros((L,), jnp.int32),
                               unroll=2)
        for b in range(ROWS):
            obuf[b, pl.ds(obase, L)] = (or_acc >> b) & 1
        return carry

    lax.fori_loop(0, OUT // L, og_body, 0)

    pltpu.sync_copy(obuf, out_hbm.at[pl.ds(base_row, ROWS)])


def kernel(x, weights):
    w_t = jnp.transpose(weights, (1, 2, 0))
    w_t = w_t.reshape(R, T // 2, 2, OUT)
    w16 = (w_t[:, :, 0, :] | (w_t[:, :, 1, :] << 16)).reshape(-1)
    mesh = plsc.VectorSubcoreMesh(core_axis_name="c", subcore_axis_name="s")
    f = functools.partial(
        pl.kernel,
        out_type=jax.ShapeDtypeStruct((B, OUT), jnp.int32),
        mesh=mesh,
        compiler_params=pltpu.CompilerParams(needs_layout_passes=False),
        scratch_types=[
            pltpu.VMEM((ROWS, F), jnp.int32),
            pltpu.VMEM((R * (T // 2) * OUT, ), jnp.int32),
            pltpu.VMEM((1 + 2 * F + 15, ), jnp.int32),
            pltpu.VMEM((ROWS, OUT), jnp.int32),
            pltpu.SemaphoreType.DMA((4,)),
            pltpu.SemaphoreType.DMA,
        ],
    )(_sc_body)
    return f(x, w16).astype(bool)
